# baseline (speedup 1.0000x reference)
import jax
import jax.numpy as jnp
from jax.experimental import pallas as pl
from jax.experimental.pallas import tpu as pltpu

_D_MODEL = 4096
_N_EXPERTS = 64
_K = 8
_BLOCK = 1024
_IDX_MASK = _N_EXPERTS - 1


def _order_flip(b):
    return b ^ (jax.lax.shift_right_arithmetic(b, 31) & 0x7FFFFFFF)


def _router_body(x_ref, w_ref, ones_ref, logits_ref, probs_ref, idx_ref,
                 wts_ref):
    logits = jax.lax.dot_general(
        x_ref[...], w_ref[...], (((1,), (1,)), ((), ())),
        preferred_element_type=jnp.float32,
    )
    logits_ref[...] = logits

    cols = jax.lax.broadcasted_iota(jnp.int32, logits.shape, 1)
    bits = jax.lax.bitcast_convert_type(logits, jnp.int32)
    key = (_order_flip(bits) & ~_IDX_MASK) | (_IDX_MASK - cols)

    maxes = []
    for _ in range(_K):
        m = jnp.max(key, axis=-1, keepdims=True)
        key = jnp.where(key == m, -(2**31), key)
        maxes.append(m)
    mk = jnp.concatenate(maxes, axis=1)
    topi = _IDX_MASK - (mk & _IDX_MASK)
    tk = mk & ~_IDX_MASK
    topl = jax.lax.bitcast_convert_type(_order_flip(tk), jnp.float32)

    mrow = topl[:, :1]
    e = jnp.exp(logits - mrow)
    s = jax.lax.dot_general(
        e, ones_ref[...], (((1,), (0,)), ((), ())),
        preferred_element_type=jnp.float32,
    )
    probs_ref[...] = e * (1.0 / s)

    tv = jnp.exp(topl - mrow)
    denom = jnp.maximum(jnp.sum(tv, axis=1, keepdims=True), 1e-9)
    wts_ref[...] = tv / denom
    idx_ref[...] = topi


def kernel(x, W):
    n_tokens = x.shape[0]
    grid = (n_tokens // _BLOCK,)
    ones = jnp.ones((_N_EXPERTS, 1), jnp.float32)
    out_shape = (
        jax.ShapeDtypeStruct((n_tokens, _N_EXPERTS), jnp.float32),
        jax.ShapeDtypeStruct((n_tokens, _N_EXPERTS), jnp.float32),
        jax.ShapeDtypeStruct((n_tokens, _K), jnp.int32),
        jax.ShapeDtypeStruct((n_tokens, _K), jnp.float32),
    )
    logits, probs, topk_indices, topk_weights = pl.pallas_call(
        _router_body,
        grid=grid,
        in_specs=[
            pl.BlockSpec((_BLOCK, _D_MODEL), lambda i: (i, 0)),
            pl.BlockSpec((_N_EXPERTS, _D_MODEL), lambda i: (0, 0)),
            pl.BlockSpec((_N_EXPERTS, 1), lambda i: (0, 0)),
        ],
        out_specs=(
            pl.BlockSpec((_BLOCK, _N_EXPERTS), lambda i: (i, 0)),
            pl.BlockSpec((_BLOCK, _N_EXPERTS), lambda i: (i, 0)),
            pl.BlockSpec((_BLOCK, _K), lambda i: (i, 0)),
            pl.BlockSpec((_BLOCK, _K), lambda i: (i, 0)),
        ),
        out_shape=out_shape,
        compiler_params=pltpu.CompilerParams(
            dimension_semantics=("parallel",),
        ),
    )(x, W, ones)
    return (logits, probs, topk_indices, topk_weights)


# speedup vs baseline: 1.1948x; 1.1948x over previous
import jax
import jax.numpy as jnp
from jax.experimental import pallas as pl
from jax.experimental.pallas import tpu as pltpu

_D_MODEL = 4096
_N_EXPERTS = 64
_K = 8
_BLOCK = 1024
_IDX_MASK = _N_EXPERTS - 1


def _router_body(x_ref, w_ref, ones_ref, logits_ref, probs_ref, idx_ref,
                 wts_ref):
    logits = jax.lax.dot_general(
        x_ref[...], w_ref[...], (((1,), (1,)), ((), ())),
        preferred_element_type=jnp.float32,
    )
    logits_ref[...] = logits

    e = jnp.exp(logits)
    s = jax.lax.dot_general(
        e, ones_ref[...], (((1,), (0,)), ((), ())),
        preferred_element_type=jnp.float32,
    )
    probs = e * (1.0 / s)
    probs_ref[...] = probs

    cols = jax.lax.broadcasted_iota(jnp.int32, probs.shape, 1)
    bits = jax.lax.bitcast_convert_type(probs, jnp.int32)
    key = jax.lax.bitcast_convert_type(
        (bits & ~_IDX_MASK) | (_IDX_MASK - cols), jnp.float32)

    maxes = []
    for _ in range(_K):
        m = jnp.max(key, axis=-1, keepdims=True)
        key = jnp.where(key == m, -jnp.inf, key)
        maxes.append(m)
    mk = jax.lax.bitcast_convert_type(
        jnp.concatenate(maxes, axis=1), jnp.int32)
    topi = _IDX_MASK - (mk & _IDX_MASK)
    topv = jax.lax.bitcast_convert_type(mk & ~_IDX_MASK, jnp.float32)
    denom = jnp.maximum(jnp.sum(topv, axis=1, keepdims=True), 1e-9)
    wts_ref[...] = topv / denom
    idx_ref[...] = topi


def kernel(x, W):
    n_tokens = x.shape[0]
    grid = (n_tokens // _BLOCK,)
    ones = jnp.ones((_N_EXPERTS, 1), jnp.float32)
    out_shape = (
        jax.ShapeDtypeStruct((n_tokens, _N_EXPERTS), jnp.float32),
        jax.ShapeDtypeStruct((n_tokens, _N_EXPERTS), jnp.float32),
        jax.ShapeDtypeStruct((n_tokens, _K), jnp.int32),
        jax.ShapeDtypeStruct((n_tokens, _K), jnp.float32),
    )
    logits, probs, topk_indices, topk_weights = pl.pallas_call(
        _router_body,
        grid=grid,
        in_specs=[
            pl.BlockSpec((_BLOCK, _D_MODEL), lambda i: (i, 0)),
            pl.BlockSpec((_N_EXPERTS, _D_MODEL), lambda i: (0, 0)),
            pl.BlockSpec((_N_EXPERTS, 1), lambda i: (0, 0)),
        ],
        out_specs=(
            pl.BlockSpec((_BLOCK, _N_EXPERTS), lambda i: (i, 0)),
            pl.BlockSpec((_BLOCK, _N_EXPERTS), lambda i: (i, 0)),
            pl.BlockSpec((_BLOCK, _K), lambda i: (i, 0)),
            pl.BlockSpec((_BLOCK, _K), lambda i: (i, 0)),
        ),
        out_shape=out_shape,
        compiler_params=pltpu.CompilerParams(
            dimension_semantics=("parallel",),
        ),
    )(x, W, ones)
    return (logits, probs, topk_indices, topk_weights)


# speedup vs baseline: 1.2239x; 1.0243x over previous
import jax
import jax.numpy as jnp
from jax.experimental import pallas as pl
from jax.experimental.pallas import tpu as pltpu

_D_MODEL = 4096
_N_EXPERTS = 64
_K = 8
_BLOCK = 1024
_IDX_MASK = _N_EXPERTS - 1


def _router_body(x_ref, w_ref, ones_ref, logits_ref, probs_ref, idx_ref,
                 wts_ref):
    logits = jax.lax.dot_general(
        x_ref[...], w_ref[...], (((1,), (1,)), ((), ())),
        preferred_element_type=jnp.float32,
    )
    logits_ref[...] = logits

    e = jnp.exp(logits)
    s = jnp.sum(e, axis=-1, keepdims=True)
    probs = e * (1.0 / s)
    probs_ref[...] = probs

    cols = jax.lax.broadcasted_iota(jnp.int32, probs.shape, 1)
    bits = jax.lax.bitcast_convert_type(probs, jnp.int32)
    key = jax.lax.bitcast_convert_type(
        (bits & ~_IDX_MASK) | (_IDX_MASK - cols), jnp.float32)

    maxes = []
    for _ in range(_K):
        m = jnp.max(key, axis=-1, keepdims=True)
        key = jnp.where(key == m, -jnp.inf, key)
        maxes.append(m)
    mk = jax.lax.bitcast_convert_type(
        jnp.concatenate(maxes, axis=1), jnp.int32)
    topi = _IDX_MASK - (mk & _IDX_MASK)
    topv = jax.lax.bitcast_convert_type(mk & ~_IDX_MASK, jnp.float32)
    denom = jnp.maximum(jnp.sum(topv, axis=1, keepdims=True), 1e-9)
    wts_ref[...] = topv / denom
    idx_ref[...] = topi


def kernel(x, W):
    n_tokens = x.shape[0]
    grid = (n_tokens // _BLOCK,)
    ones = jnp.ones((_N_EXPERTS, 1), jnp.float32)
    out_shape = (
        jax.ShapeDtypeStruct((n_tokens, _N_EXPERTS), jnp.float32),
        jax.ShapeDtypeStruct((n_tokens, _N_EXPERTS), jnp.float32),
        jax.ShapeDtypeStruct((n_tokens, _K), jnp.int32),
        jax.ShapeDtypeStruct((n_tokens, _K), jnp.float32),
    )
    logits, probs, topk_indices, topk_weights = pl.pallas_call(
        _router_body,
        grid=grid,
        in_specs=[
            pl.BlockSpec((_BLOCK, _D_MODEL), lambda i: (i, 0)),
            pl.BlockSpec((_N_EXPERTS, _D_MODEL), lambda i: (0, 0)),
            pl.BlockSpec((_N_EXPERTS, 1), lambda i: (0, 0)),
        ],
        out_specs=(
            pl.BlockSpec((_BLOCK, _N_EXPERTS), lambda i: (i, 0)),
            pl.BlockSpec((_BLOCK, _N_EXPERTS), lambda i: (i, 0)),
            pl.BlockSpec((_BLOCK, _K), lambda i: (i, 0)),
            pl.BlockSpec((_BLOCK, _K), lambda i: (i, 0)),
        ),
        out_shape=out_shape,
        compiler_params=pltpu.CompilerParams(
            dimension_semantics=("parallel",),
        ),
    )(x, W, ones)
    return (logits, probs, topk_indices, topk_weights)


# speedup vs baseline: 1.2250x; 1.0009x over previous
import jax
import jax.numpy as jnp
from jax.experimental import pallas as pl
from jax.experimental.pallas import tpu as pltpu

_D_MODEL = 4096
_N_EXPERTS = 64
_K = 8
_BLOCK = 1024
_IDX_MASK = _N_EXPERTS - 1


def _router_body(x_ref, w_ref, logits_ref, probs_ref, idx_ref, wts_ref):
    logits = jax.lax.dot_general(
        x_ref[...], w_ref[...], (((1,), (1,)), ((), ())),
        preferred_element_type=jnp.float32,
    )
    logits_ref[...] = logits

    e = jnp.exp(logits)
    s = jnp.sum(e, axis=-1, keepdims=True)
    probs = e * (1.0 / s)
    probs_ref[...] = probs

    cols = jax.lax.broadcasted_iota(jnp.int32, probs.shape, 1)
    bits = jax.lax.bitcast_convert_type(probs, jnp.int32)
    key = jax.lax.bitcast_convert_type(
        (bits & ~_IDX_MASK) | (_IDX_MASK - cols), jnp.float32)

    maxes = []
    for _ in range(_K):
        m = jnp.max(key, axis=-1, keepdims=True)
        key = jnp.where(key == m, -jnp.inf, key)
        maxes.append(m)
    mk = jax.lax.bitcast_convert_type(
        jnp.concatenate(maxes, axis=1), jnp.int32)
    topi = _IDX_MASK - (mk & _IDX_MASK)
    topv = jax.lax.bitcast_convert_type(mk & ~_IDX_MASK, jnp.float32)
    denom = jnp.maximum(jnp.sum(topv, axis=1, keepdims=True), 1e-9)
    wts_ref[...] = topv / denom
    idx_ref[...] = topi


def kernel(x, W):
    n_tokens = x.shape[0]
    grid = (n_tokens // _BLOCK,)
    out_shape = (
        jax.ShapeDtypeStruct((n_tokens, _N_EXPERTS), jnp.float32),
        jax.ShapeDtypeStruct((n_tokens, _N_EXPERTS), jnp.float32),
        jax.ShapeDtypeStruct((n_tokens, _K), jnp.int32),
        jax.ShapeDtypeStruct((n_tokens, _K), jnp.float32),
    )
    logits, probs, topk_indices, topk_weights = pl.pallas_call(
        _router_body,
        grid=grid,
        in_specs=[
            pl.BlockSpec((_BLOCK, _D_MODEL), lambda i: (i, 0)),
            pl.BlockSpec((_N_EXPERTS, _D_MODEL), lambda i: (0, 0)),
        ],
        out_specs=(
            pl.BlockSpec((_BLOCK, _N_EXPERTS), lambda i: (i, 0)),
            pl.BlockSpec((_BLOCK, _N_EXPERTS), lambda i: (i, 0)),
            pl.BlockSpec((_BLOCK, _K), lambda i: (i, 0)),
            pl.BlockSpec((_BLOCK, _K), lambda i: (i, 0)),
        ),
        out_shape=out_shape,
        compiler_params=pltpu.CompilerParams(
            dimension_semantics=("parallel",),
        ),
    )(x, W)
    return (logits, probs, topk_indices, topk_weights)
